# baseline (speedup 1.0000x reference)
import jax
import jax.numpy as jnp
from jax import lax
from jax.experimental import pallas as pl
from jax.experimental.pallas import tpu as pltpu
from jax.experimental.pallas import tpu_sc as plsc

C = 4
L = 50
P = 4
N = 10000
E = 320000
D = 128

NC = 2
NS = 16
CK = 80
NCHUNK = 252
EPT = NCHUNK * CK
E_PAD = NS * EPT
NACC = 10240
RPT = NACC // NS
NB = 3
NGA = 3
CKA = NGA * 16
CKB = CK - CKA


def _body(x_hbm, rec_hbm, w_hbm, b2d_hbm, lbl_hbm,
          out_hbm,
          lblpk_v, wtab_v, recs, srcs, dstsA, dstsB, rows, acc,
          semG, semSA, semSB, semR, stag):
    core = lax.axis_index("c")
    tid = lax.axis_index("s")

    pltpu.sync_copy(lbl_hbm, lblpk_v)

    def labels_of(i16):
        word = plsc.load_gather(lblpk_v, [lax.shift_right_logical(i16, 1)])
        sh = lax.shift_left(jnp.bitwise_and(i16, 1), 4)
        return jnp.bitwise_and(lax.shift_right_logical(word, sh), 0xFFFF)

    for pass_i in range(2):
        ch = core * 2 + pass_i
        pltpu.sync_copy(w_hbm.at[pl.ds(ch * (L * L * P), L * L * P)], wtab_v)

        for g in range(RPT // CK):
            row_base = tid * RPT + g * CK
            for k in range(CK // 16):
                i16 = jnp.arange(16, dtype=jnp.int32) + (row_base + k * 16)
                lbl16 = labels_of(i16)
                srcs[0][pl.ds(k * 16, 16)] = lbl16 + ch * L
            pltpu.async_copy(b2d_hbm.at[srcs[0]], rows[0].at[pl.ds(0, CK)],
                             semG[0]).wait()
            pltpu.sync_copy(rows[0], acc.at[pl.ds(row_base, CK)])

        def rec_dma(g, b):
            return pltpu.async_copy(
                rec_hbm.at[pl.ds(tid * EPT + g * CK, CK)], recs[b], semR[b])

        def rec_wait(g, b):
            pltpu.make_async_copy(
                rec_hbm.at[pl.ds(tid * EPT + g * CK, CK)], recs[b],
                semR[b]).wait()

        def unpack(b):
            for k in range(CK // 16):
                r16 = recs[b][pl.ds(k * 16, 16)]
                srcs[b][pl.ds(k * 16, 16)] = jnp.bitwise_and(r16, 0x3FFF)
                d16 = jnp.bitwise_and(lax.shift_right_logical(r16, 14),
                                      0x3FFF)
                if k < NGA:
                    dstsA[b][pl.ds(k * 16, 16)] = d16
                else:
                    dstsB[b][pl.ds((k - NGA) * 16, 16)] = d16

        def gather(b):
            return pltpu.async_copy(x_hbm.at[srcs[b]], rows[b], semG[b])

        def gather_wait(b):
            pltpu.make_async_copy(x_hbm.at[srcs[b]], rows[b], semG[b]).wait()

        def scatter_a(b):
            return pltpu.async_copy(rows[b].at[pl.ds(0, CKA)],
                                    acc.at[dstsA[b]], semSA[b], add=True)

        def scatter_b(b):
            return pltpu.async_copy(rows[b].at[pl.ds(CKA, CKB)],
                                    acc.at[dstsB[b]], semSB[b], add=True)

        def scatter_wait(b):
            pltpu.make_async_copy(rows[b].at[pl.ds(0, CKA)],
                                  acc.at[dstsA[b]], semSA[b]).wait()
            pltpu.make_async_copy(rows[b].at[pl.ds(CKA, CKB)],
                                  acc.at[dstsB[b]], semSB[b]).wait()

        def scale_grp(b, k16, dref, doff):
            s16 = srcs[b][pl.ds(k16 * 16, 16)]
            d16 = dref[pl.ds(doff * 16, 16)]
            p16 = lax.shift_right_logical(
                recs[b][pl.ds(k16 * 16, 16)], 28)
            li = labels_of(d16)
            lj = labels_of(s16)
            w16 = plsc.load_gather(wtab_v, [(li * L + lj) * P + p16])
            for j in range(16):
                ws = w16[j]
                for sblk in range(D // 16):
                    sl = (k16 * 16 + j, pl.ds(sblk * 16, 16))
                    rows[b][sl] = rows[b][sl] * ws

        def scale_a(b):
            def grp(k16, c2):
                scale_grp(b, k16, dstsA[b], k16)
                return c2
            lax.fori_loop(0, NGA, grp, None)

        def scale_b(b):
            def grp(k16, c2):
                scale_grp(b, k16, dstsB[b], k16 - NGA)
                return c2
            lax.fori_loop(NGA, CK // 16, grp, None)

        rec_dma(0, 0).wait()
        unpack(0)
        gather(0)
        rec_dma(1, 1).wait()
        unpack(1)
        gather(1)
        rec_dma(2, 2)
        plsc.subcore_barrier()
        spin = lax.fori_loop(0, tid * 400, lambda _, c: c + 1, pass_i)
        stag[0] = spin

        NI = NCHUNK // NB

        def ring_body(i, carry):
            for k in range(NB):
                b = k
                b2 = (k + 2) % NB
                g = NB * i + k
                gather_wait(b)
                scale_a(b)
                scatter_a(b)
                scale_b(b)
                scatter_b(b)
                def prep(first):
                    rec_wait(g + 2, b2)
                    if first:
                        @pl.when(i >= 1)
                        def _():
                            scatter_wait(b2)
                    else:
                        scatter_wait(b2)
                    unpack(b2)
                    gather(b2)

                if k == 0:
                    prep(first=True)
                    @pl.when(i < NI - 1)
                    def _():
                        rec_dma(g + 3, k)
                else:
                    @pl.when(i < NI - 1)
                    def _():
                        prep(first=False)
                        rec_dma(g + 3, k)
            return carry

        lax.fori_loop(0, NI, ring_body, None)
        for b in range(NB):
            scatter_wait(b)
        plsc.subcore_barrier()

        pltpu.sync_copy(acc.at[pl.ds(tid * RPT, RPT)],
                        out_hbm.at[pl.ds(ch * NACC + tid * RPT, RPT)])
        plsc.subcore_barrier()


@jax.jit
def _run(x_p, rec_p, w_flat, b2d, lbl_pk):
    mesh = plsc.VectorSubcoreMesh(core_axis_name="c", subcore_axis_name="s",
                                  num_cores=NC, num_subcores=NS)
    return pl.kernel(
        _body,
        out_type=jax.ShapeDtypeStruct((C * NACC, D), jnp.float32),
        mesh=mesh,
        compiler_params=pltpu.CompilerParams(needs_layout_passes=False),
        scratch_types=[
            pltpu.VMEM((NACC // 2,), jnp.int32),
            pltpu.VMEM((L * L * P,), jnp.float32),
            [pltpu.VMEM((CK,), jnp.int32) for _ in range(NB)],
            [pltpu.VMEM((CK,), jnp.int32) for _ in range(NB)],
            [pltpu.VMEM((CKA,), jnp.int32) for _ in range(NB)],
            [pltpu.VMEM((CKB,), jnp.int32) for _ in range(NB)],
            [pltpu.VMEM((CK, D), jnp.float32) for _ in range(NB)],
            pltpu.VMEM_SHARED((NACC, D), jnp.float32),
            [pltpu.SemaphoreType.DMA for _ in range(NB)],
            [pltpu.SemaphoreType.DMA for _ in range(NB)],
            [pltpu.SemaphoreType.DMA for _ in range(NB)],
            [pltpu.SemaphoreType.DMA for _ in range(NB)],
            pltpu.SMEM((1,), jnp.int32),
        ],
    )(x_p, rec_p, w_flat, b2d, lbl_pk)


def kernel(x, edge_index, node_labels, edge_prop, Param_W, Param_b):
    src = edge_index[0]
    dst = edge_index[1]
    pad = E_PAD - E
    rec = src | (dst << 14) | (edge_prop << 28)
    rec_p = jnp.concatenate([rec, jnp.full((pad,), N, jnp.int32)])
    x_p = jnp.concatenate([x, jnp.zeros((8, D), jnp.float32)])
    lbl_full = jnp.concatenate(
        [node_labels, jnp.zeros((NACC - N,), jnp.int32)])
    lbl2 = lbl_full.reshape(NACC // 2, 2)
    lbl_pk = lbl2[:, 0] | (lbl2[:, 1] << 16)
    b2d = Param_b.reshape(C * L, D)
    out = _run(x_p, rec_p, Param_W, b2d, lbl_pk)
    return out.reshape(C, NACC, D)[:, :N]


# speedup vs baseline: 1.0027x; 1.0027x over previous
import jax
import jax.numpy as jnp
from jax import lax
from jax.experimental import pallas as pl
from jax.experimental.pallas import tpu as pltpu
from jax.experimental.pallas import tpu_sc as plsc

C = 4
L = 50
P = 4
N = 10000
E = 320000
D = 128

NC = 2
NS = 16
CK = 80
NCHUNK = 252
EPT = NCHUNK * CK
E_PAD = NS * EPT
NACC = 10240
RPT = NACC // NS
NB = 3


def _body(x_hbm, rec_hbm, w_hbm, b2d_hbm, lbl_hbm,
          out_hbm,
          lblpk_v, wtab_v, recs, srcs, dsts, rows, acc, semG, semS, semR):
    core = lax.axis_index("c")
    tid = lax.axis_index("s")

    pltpu.sync_copy(lbl_hbm, lblpk_v)

    def labels_of(i16):
        word = plsc.load_gather(lblpk_v, [lax.shift_right_logical(i16, 1)])
        sh = lax.shift_left(jnp.bitwise_and(i16, 1), 4)
        return jnp.bitwise_and(lax.shift_right_logical(word, sh), 0xFFFF)

    for pass_i in range(2):
        ch = core * 2 + pass_i
        pltpu.sync_copy(w_hbm.at[pl.ds(ch * (L * L * P), L * L * P)], wtab_v)

        for g in range(RPT // CK):
            row_base = tid * RPT + g * CK
            for k in range(CK // 16):
                i16 = jnp.arange(16, dtype=jnp.int32) + (row_base + k * 16)
                lbl16 = labels_of(i16)
                srcs[0][pl.ds(k * 16, 16)] = lbl16 + ch * L
            pltpu.async_copy(b2d_hbm.at[srcs[0]], rows[0].at[pl.ds(0, CK)],
                             semG[0]).wait()
            pltpu.sync_copy(rows[0], acc.at[pl.ds(row_base, CK)])

        def rec_dma(g, b):
            return pltpu.async_copy(
                rec_hbm.at[pl.ds(tid * EPT + g * CK, CK)], recs[b], semR[b])

        def rec_wait(g, b):
            pltpu.make_async_copy(
                rec_hbm.at[pl.ds(tid * EPT + g * CK, CK)], recs[b],
                semR[b]).wait()

        def unpack(b):
            for k in range(CK // 16):
                r16 = recs[b][pl.ds(k * 16, 16)]
                srcs[b][pl.ds(k * 16, 16)] = jnp.bitwise_and(r16, 0x3FFF)
                dsts[b][pl.ds(k * 16, 16)] = jnp.bitwise_and(
                    lax.shift_right_logical(r16, 14), 0x3FFF)

        def gather(b):
            return pltpu.async_copy(x_hbm.at[srcs[b]], rows[b], semG[b])

        def gather_wait(b):
            pltpu.make_async_copy(x_hbm.at[srcs[b]], rows[b], semG[b]).wait()

        def scatter(b):
            return pltpu.async_copy(rows[b], acc.at[dsts[b]], semS[b],
                                    add=True)

        def scatter_wait(b):
            pltpu.make_async_copy(rows[b], acc.at[dsts[b]], semS[b]).wait()

        def scale(b):
            def grp(k16, c2):
                s16 = srcs[b][pl.ds(k16 * 16, 16)]
                d16 = dsts[b][pl.ds(k16 * 16, 16)]
                p16 = lax.shift_right_logical(
                    recs[b][pl.ds(k16 * 16, 16)], 28)
                li = labels_of(d16)
                lj = labels_of(s16)
                w16 = plsc.load_gather(wtab_v, [(li * L + lj) * P + p16])
                for j in range(16):
                    ws = w16[j]
                    for sblk in range(D // 16):
                        sl = (k16 * 16 + j, pl.ds(sblk * 16, 16))
                        rows[b][sl] = rows[b][sl] * ws
                return c2
            lax.fori_loop(0, CK // 16, grp, None)

        rec_dma(0, 0).wait()
        unpack(0)
        gather(0)
        rec_dma(1, 1).wait()
        unpack(1)
        gather(1)
        rec_dma(2, 2)
        plsc.subcore_barrier()

        NI = NCHUNK // NB

        def ring_body(i, carry):
            for k in range(NB):
                b = k
                b2 = (k + 2) % NB
                g = NB * i + k
                gather_wait(b)
                scale(b)
                scatter(b)
                def prep(first):
                    rec_wait(g + 2, b2)
                    if first:
                        @pl.when(i >= 1)
                        def _():
                            scatter_wait(b2)
                    else:
                        scatter_wait(b2)
                    unpack(b2)
                    gather(b2)

                if k == 0:
                    prep(first=True)
                    @pl.when(i < NI - 1)
                    def _():
                        rec_dma(g + 3, k)
                else:
                    @pl.when(i < NI - 1)
                    def _():
                        prep(first=False)
                        rec_dma(g + 3, k)
            return carry

        lax.fori_loop(0, NI, ring_body, None)
        for b in range(NB):
            scatter_wait(b)
        plsc.subcore_barrier()

        pltpu.sync_copy(acc.at[pl.ds(tid * RPT, RPT)],
                        out_hbm.at[pl.ds(ch * NACC + tid * RPT, RPT)])
        plsc.subcore_barrier()


@jax.jit
def _run(x_p, rec_p, w_flat, b2d, lbl_pk):
    mesh = plsc.VectorSubcoreMesh(core_axis_name="c", subcore_axis_name="s",
                                  num_cores=NC, num_subcores=NS)
    return pl.kernel(
        _body,
        out_type=jax.ShapeDtypeStruct((C * NACC, D), jnp.float32),
        mesh=mesh,
        compiler_params=pltpu.CompilerParams(needs_layout_passes=False),
        scratch_types=[
            pltpu.VMEM((NACC // 2,), jnp.int32),
            pltpu.VMEM((L * L * P,), jnp.float32),
            [pltpu.VMEM((CK,), jnp.int32) for _ in range(NB)],
            [pltpu.VMEM((CK,), jnp.int32) for _ in range(NB)],
            [pltpu.VMEM((CK,), jnp.int32) for _ in range(NB)],
            [pltpu.VMEM((CK, D), jnp.float32) for _ in range(NB)],
            pltpu.VMEM_SHARED((NACC, D), jnp.float32),
            [pltpu.SemaphoreType.DMA for _ in range(NB)],
            [pltpu.SemaphoreType.DMA for _ in range(NB)],
            [pltpu.SemaphoreType.DMA for _ in range(NB)],
        ],
    )(x_p, rec_p, w_flat, b2d, lbl_pk)


def kernel(x, edge_index, node_labels, edge_prop, Param_W, Param_b):
    src = edge_index[0]
    dst = edge_index[1]
    pad = E_PAD - E
    rec = src | (dst << 14) | (edge_prop << 28)
    rec_p = jnp.concatenate([rec, jnp.full((pad,), N, jnp.int32)])
    x_p = jnp.concatenate([x, jnp.zeros((8, D), jnp.float32)])
    lbl_full = jnp.concatenate(
        [node_labels, jnp.zeros((NACC - N,), jnp.int32)])
    lbl2 = lbl_full.reshape(NACC // 2, 2)
    lbl_pk = lbl2[:, 0] | (lbl2[:, 1] << 16)
    b2d = Param_b.reshape(C * L, D)
    out = _run(x_p, rec_p, Param_W, b2d, lbl_pk)
    return out.reshape(C, NACC, D)[:, :N]
